# baseline (speedup 1.0000x reference)
import functools

import jax
import jax.numpy as jnp
from jax import lax
from jax.experimental import pallas as pl
from jax.experimental.pallas import tpu as pltpu
from jax.experimental.pallas import tpu_sc as plsc

G = 64
NC = 2
NS = 16
CHUNK = 128
KBUF = 8
PLAG = 4


def _make_sc_segment_sum(n_nodes, feat, n_chunk_rows):
    fh = feat // NC
    cpw = n_chunk_rows // NS
    assert cpw % KBUF == 0 and cpw >= 2 * KBUF
    stripe = ((n_nodes + NS - 1) // NS + 7) // 8 * 8
    n_acc = NS * stripe
    mesh = plsc.VectorSubcoreMesh(core_axis_name="c", subcore_axis_name="s")

    @functools.partial(
        pl.kernel,
        out_type=jax.ShapeDtypeStruct((NC, n_acc, fh), jnp.float32),
        mesh=mesh,
        scratch_types=[
            pltpu.VMEM((cpw, CHUNK), jnp.int32),
            pltpu.VMEM((cpw, CHUNK), jnp.int32),
            [pltpu.VMEM((CHUNK, fh), jnp.float32) for _ in range(KBUF)],
            pltpu.VMEM_SHARED((n_acc, fh), jnp.float32),
            pltpu.VMEM_SHARED((n_acc, fh), jnp.float32),
            pltpu.SemaphoreType.DMA((KBUF,)),
            pltpu.SemaphoreType.DMA((KBUF,)),
        ],
        compiler_params=pltpu.CompilerParams(use_tc_tiling_on_sc=False),
    )
    def seg_sum(h_hbm, src_hbm, dst_hbm, zero_hbm, out_hbm,
                src_v, dst_v, rows, acc, h_sh, sem_g, sem_s):
        cid = lax.axis_index("c")
        sid = lax.axis_index("s")
        base = sid * cpw

        pltpu.sync_copy(src_hbm.at[pl.ds(base, cpw)], src_v)
        pltpu.sync_copy(dst_hbm.at[pl.ds(base, cpw)], dst_v)
        pltpu.sync_copy(zero_hbm, acc.at[pl.ds(sid * stripe, stripe)])
        pltpu.sync_copy(h_hbm.at[cid, pl.ds(sid * stripe, stripe)],
                        h_sh.at[pl.ds(sid * stripe, stripe)])

        plsc.subcore_barrier()

        for b in range(KBUF):
            pltpu.async_copy(h_sh.at[src_v.at[b]], rows[b], sem_g.at[b])

        def step(i, b, wait_prev, prefetch):
            pltpu.make_async_copy(
                h_sh.at[src_v.at[i]], rows[b], sem_g.at[b]).wait()
            pltpu.async_copy(rows[b], acc.at[dst_v.at[i]], sem_s.at[b],
                             add=True)
            pb = (b - PLAG) % KBUF
            if wait_prev:
                pltpu.make_async_copy(
                    rows[pb], acc.at[dst_v.at[i]], sem_s.at[pb]).wait()
            if prefetch:
                pltpu.async_copy(h_sh.at[src_v.at[i - PLAG + KBUF]],
                                 rows[pb], sem_g.at[pb])

        for b in range(KBUF):
            step(b, b, wait_prev=b >= PLAG, prefetch=b >= PLAG)

        def body(k, carry):
            for b in range(KBUF):
                step(k * KBUF + b, b, wait_prev=True, prefetch=True)
            return carry
        lax.fori_loop(1, cpw // KBUF - 1, body, 0)

        for b in range(KBUF):
            i = cpw - KBUF + b
            step(i, b, wait_prev=True, prefetch=b < PLAG)

        for j in range(cpw - PLAG, cpw):
            b = j % KBUF
            pltpu.make_async_copy(
                rows[b], acc.at[dst_v.at[0]], sem_s.at[b]).wait()

        plsc.subcore_barrier()
        pltpu.sync_copy(acc.at[pl.ds(sid * stripe, stripe)],
                        out_hbm.at[cid, pl.ds(sid * stripe, stripe)])

    return seg_sum, n_acc, stripe


def _linear_relu_body(x_ref, w_ref, b_ref, o_ref, h2_ref):
    h = jnp.maximum(
        jnp.dot(x_ref[...], w_ref[...], preferred_element_type=jnp.float32)
        + b_ref[...], 0.0)
    o_ref[...] = h
    n, f = h.shape
    h2_ref[0, :n, :] = h[:, :f // 2]
    h2_ref[1, :n, :] = h[:, f // 2:]


def _mlp_body(n_nodes, h_ref, p_ref, w1_ref, b1_ref, w2_ref, b2_ref,
              o_ref, h2_ref):
    agg = jnp.concatenate([p_ref[0, :n_nodes, :], p_ref[1, :n_nodes, :]],
                          axis=1)
    z = h_ref[...] + agg
    a = jnp.maximum(
        jnp.dot(z, w1_ref[...], preferred_element_type=jnp.float32)
        + b1_ref[...], 0.0)
    h = jnp.maximum(
        jnp.dot(a, w2_ref[...], preferred_element_type=jnp.float32)
        + b2_ref[...], 0.0)
    o_ref[...] = h
    n, f = h.shape
    h2_ref[0, :n, :] = h[:, :f // 2]
    h2_ref[1, :n, :] = h[:, f // 2:]


def _mlp_pool_body(n_nodes, h_ref, p_ref, w1_ref, b1_ref, w2_ref, b2_ref,
                   batch_ref, o_ref):
    agg = jnp.concatenate([p_ref[0, :n_nodes, :], p_ref[1, :n_nodes, :]],
                          axis=1)
    z = h_ref[...] + agg
    a = jnp.maximum(
        jnp.dot(z, w1_ref[...], preferred_element_type=jnp.float32)
        + b1_ref[...], 0.0)
    h2 = jnp.maximum(
        jnp.dot(a, w2_ref[...], preferred_element_type=jnp.float32)
        + b2_ref[...], 0.0)
    gids = lax.broadcasted_iota(jnp.int32, (n_nodes, G), 1)
    onehot = (batch_ref[...] == gids).astype(jnp.float32)
    sums = lax.dot_general(onehot, h2, (((0,), (0,)), ((), ())),
                           preferred_element_type=jnp.float32)
    counts = jnp.sum(onehot, axis=0)
    o_ref[...] = sums / jnp.maximum(counts, 1.0)[:, None]


def kernel(x, edge_index, batch, W_in, b_in,
           W1_0, b1_0, W2_0, b2_0, W1_1, b1_1, W2_1, b2_1):
    x = x.astype(jnp.float32)
    n, _ = x.shape
    h_dim = W_in.shape[1]
    e = edge_index.shape[1]

    per_tile = NS * KBUF * CHUNK
    e_pad = -(-e // per_tile) * per_tile
    n_chunk_rows = e_pad // CHUNK
    seg_sum, n_acc, stripe = _make_sc_segment_sum(n, h_dim, n_chunk_rows)
    fh = h_dim // NC

    pad = e_pad - e
    src2d = jnp.concatenate(
        [edge_index[0], jnp.zeros((pad,), jnp.int32)]).reshape(-1, CHUNK)
    dst2d = jnp.concatenate(
        [edge_index[1], jnp.full((pad,), n_acc - 1, jnp.int32)]
    ).reshape(-1, CHUNK)
    zeros = jnp.zeros((stripe, fh), jnp.float32)
    batch2d = batch.reshape(n, 1)
    b_in2 = b_in.reshape(1, -1)
    b1_02, b2_02 = b1_0.reshape(1, -1), b2_0.reshape(1, -1)
    b1_12, b2_12 = b1_1.reshape(1, -1), b2_1.reshape(1, -1)

    h_split = jax.ShapeDtypeStruct((NC, n_acc, fh), jnp.float32)

    h0, h0s = pl.pallas_call(
        _linear_relu_body,
        out_shape=(jax.ShapeDtypeStruct((n, h_dim), jnp.float32), h_split),
    )(x, W_in, b_in2)

    p0 = seg_sum(h0s, src2d, dst2d, zeros)

    h1, h1s = pl.pallas_call(
        functools.partial(_mlp_body, n),
        out_shape=(jax.ShapeDtypeStruct((n, h_dim), jnp.float32), h_split),
    )(h0, p0, W1_0, b1_02, W2_0, b2_02)

    p1 = seg_sum(h1s, src2d, dst2d, zeros)

    out = pl.pallas_call(
        functools.partial(_mlp_pool_body, n),
        out_shape=jax.ShapeDtypeStruct((G, h_dim), jnp.float32),
    )(h1, p1, W1_1, b1_12, W2_1, b2_12, batch2d)

    return out


# speedup vs baseline: 1.0358x; 1.0358x over previous
import functools

import jax
import jax.numpy as jnp
from jax import lax
from jax.experimental import pallas as pl
from jax.experimental.pallas import tpu as pltpu
from jax.experimental.pallas import tpu_sc as plsc

G = 64
NC = 2
NS = 16
CHUNK = 128
KBUF = 8
PLAG = 4


def _make_sc_segment_sum(n_nodes, feat, n_chunk_rows):
    fh = feat // NC
    cpw = n_chunk_rows // NS
    assert cpw % KBUF == 0 and cpw >= 2 * KBUF
    stripe = ((n_nodes + NS - 1) // NS + 7) // 8 * 8
    n_acc = NS * stripe
    mesh = plsc.VectorSubcoreMesh(core_axis_name="c", subcore_axis_name="s")

    @functools.partial(
        pl.kernel,
        out_type=jax.ShapeDtypeStruct((NC, n_acc, fh), jnp.float32),
        mesh=mesh,
        scratch_types=[
            pltpu.VMEM((cpw, CHUNK), jnp.int32),
            pltpu.VMEM((cpw, CHUNK), jnp.int32),
            [pltpu.VMEM((CHUNK, fh), jnp.float32) for _ in range(KBUF)],
            pltpu.VMEM_SHARED((n_acc, fh), jnp.float32),
            pltpu.VMEM_SHARED((n_acc, fh), jnp.float32),
            pltpu.SemaphoreType.DMA((KBUF,)),
            pltpu.SemaphoreType.DMA((KBUF,)),
        ],
        compiler_params=pltpu.CompilerParams(use_tc_tiling_on_sc=False),
    )
    def seg_sum(h_hbm, src_hbm, dst_hbm, out_hbm,
                src_v, dst_v, rows, acc, h_sh, sem_g, sem_s):
        cid = lax.axis_index("c")
        sid = lax.axis_index("s")
        base = sid * cpw

        pltpu.sync_copy(src_hbm.at[pl.ds(base, cpw)], src_v)
        pltpu.sync_copy(dst_hbm.at[pl.ds(base, cpw)], dst_v)
        pltpu.sync_copy(h_hbm.at[cid, pl.ds(sid * stripe, stripe)],
                        acc.at[pl.ds(sid * stripe, stripe)])
        pltpu.sync_copy(h_hbm.at[cid, pl.ds(sid * stripe, stripe)],
                        h_sh.at[pl.ds(sid * stripe, stripe)])

        plsc.subcore_barrier()

        for b in range(KBUF):
            pltpu.async_copy(h_sh.at[src_v.at[b]], rows[b], sem_g.at[b])

        def step(i, b, wait_prev, prefetch):
            pltpu.make_async_copy(
                h_sh.at[src_v.at[i]], rows[b], sem_g.at[b]).wait()
            pltpu.async_copy(rows[b], acc.at[dst_v.at[i]], sem_s.at[b],
                             add=True)
            pb = (b - PLAG) % KBUF
            if wait_prev:
                pltpu.make_async_copy(
                    rows[pb], acc.at[dst_v.at[i]], sem_s.at[pb]).wait()
            if prefetch:
                pltpu.async_copy(h_sh.at[src_v.at[i - PLAG + KBUF]],
                                 rows[pb], sem_g.at[pb])

        for b in range(KBUF):
            step(b, b, wait_prev=b >= PLAG, prefetch=b >= PLAG)

        def body(k, carry):
            for b in range(KBUF):
                step(k * KBUF + b, b, wait_prev=True, prefetch=True)
            return carry
        lax.fori_loop(1, cpw // KBUF - 1, body, 0)

        for b in range(KBUF):
            i = cpw - KBUF + b
            step(i, b, wait_prev=True, prefetch=b < PLAG)

        for j in range(cpw - PLAG, cpw):
            b = j % KBUF
            pltpu.make_async_copy(
                rows[b], acc.at[dst_v.at[0]], sem_s.at[b]).wait()

        plsc.subcore_barrier()
        pltpu.sync_copy(acc.at[pl.ds(sid * stripe, stripe)],
                        out_hbm.at[cid, pl.ds(sid * stripe, stripe)])

    return seg_sum, n_acc, stripe


def _linear_relu_body(x_ref, w_ref, b_ref, h2_ref):
    h = jnp.maximum(
        jnp.dot(x_ref[...], w_ref[...], preferred_element_type=jnp.float32)
        + b_ref[...], 0.0)
    n, f = h.shape
    h2_ref[0, :n, :] = h[:, :f // 2]
    h2_ref[1, :n, :] = h[:, f // 2:]


def _mlp_body(n_nodes, p_ref, w1_ref, b1_ref, w2_ref, b2_ref, h2_ref):
    z = jnp.concatenate([p_ref[0, :n_nodes, :], p_ref[1, :n_nodes, :]],
                        axis=1)
    a = jnp.maximum(
        jnp.dot(z, w1_ref[...], preferred_element_type=jnp.float32)
        + b1_ref[...], 0.0)
    h = jnp.maximum(
        jnp.dot(a, w2_ref[...], preferred_element_type=jnp.float32)
        + b2_ref[...], 0.0)
    n, f = h.shape
    h2_ref[0, :n, :] = h[:, :f // 2]
    h2_ref[1, :n, :] = h[:, f // 2:]


def _mlp_pool_body(n_nodes, p_ref, w1_ref, b1_ref, w2_ref, b2_ref,
                   batch_ref, o_ref):
    z = jnp.concatenate([p_ref[0, :n_nodes, :], p_ref[1, :n_nodes, :]],
                        axis=1)
    a = jnp.maximum(
        jnp.dot(z, w1_ref[...], preferred_element_type=jnp.float32)
        + b1_ref[...], 0.0)
    h2 = jnp.maximum(
        jnp.dot(a, w2_ref[...], preferred_element_type=jnp.float32)
        + b2_ref[...], 0.0)
    gids = lax.broadcasted_iota(jnp.int32, (n_nodes, G), 1)
    onehot = (batch_ref[...] == gids).astype(jnp.float32)
    sums = lax.dot_general(onehot, h2, (((0,), (0,)), ((), ())),
                           preferred_element_type=jnp.float32)
    counts = jnp.sum(onehot, axis=0)
    o_ref[...] = sums / jnp.maximum(counts, 1.0)[:, None]


def kernel(x, edge_index, batch, W_in, b_in,
           W1_0, b1_0, W2_0, b2_0, W1_1, b1_1, W2_1, b2_1):
    x = x.astype(jnp.float32)
    n, _ = x.shape
    h_dim = W_in.shape[1]
    e = edge_index.shape[1]

    per_tile = NS * KBUF * CHUNK
    e_pad = -(-e // per_tile) * per_tile
    n_chunk_rows = e_pad // CHUNK
    seg_sum, n_acc, stripe = _make_sc_segment_sum(n, h_dim, n_chunk_rows)
    fh = h_dim // NC

    pad = e_pad - e
    src2d = jnp.concatenate(
        [edge_index[0], jnp.zeros((pad,), jnp.int32)]).reshape(-1, CHUNK)
    dst2d = jnp.concatenate(
        [edge_index[1], jnp.full((pad,), n_acc - 1, jnp.int32)]
    ).reshape(-1, CHUNK)
    batch2d = batch.reshape(n, 1)
    b_in2 = b_in.reshape(1, -1)
    b1_02, b2_02 = b1_0.reshape(1, -1), b2_0.reshape(1, -1)
    b1_12, b2_12 = b1_1.reshape(1, -1), b2_1.reshape(1, -1)

    h_split = jax.ShapeDtypeStruct((NC, n_acc, fh), jnp.float32)

    h0s = pl.pallas_call(
        _linear_relu_body, out_shape=h_split,
    )(x, W_in, b_in2)

    z0 = seg_sum(h0s, src2d, dst2d)

    h1s = pl.pallas_call(
        functools.partial(_mlp_body, n), out_shape=h_split,
    )(z0, W1_0, b1_02, W2_0, b2_02)

    z1 = seg_sum(h1s, src2d, dst2d)

    out = pl.pallas_call(
        functools.partial(_mlp_pool_body, n),
        out_shape=jax.ShapeDtypeStruct((G, h_dim), jnp.float32),
    )(z1, W1_1, b1_12, W2_1, b2_12, batch2d)

    return out


# speedup vs baseline: 1.1023x; 1.0642x over previous
import functools

import jax
import jax.numpy as jnp
from jax import lax
from jax.experimental import pallas as pl
from jax.experimental.pallas import tpu as pltpu
from jax.experimental.pallas import tpu_sc as plsc

G = 64
NC = 2
NS = 16
CHUNK = 128
KBUF = 8
PLAG = 4


def _make_sc_segment_sum(n_nodes, feat, n_chunk_rows):
    fh = feat // NC
    cpw = n_chunk_rows // NS
    assert cpw % KBUF == 0 and cpw >= 2 * KBUF
    stripe = ((n_nodes + NS - 1) // NS + 7) // 8 * 8
    n_acc = NS * stripe
    mesh = plsc.VectorSubcoreMesh(core_axis_name="c", subcore_axis_name="s")

    @functools.partial(
        pl.kernel,
        out_type=jax.ShapeDtypeStruct((NC, n_acc, fh), jnp.float32),
        mesh=mesh,
        scratch_types=[
            pltpu.VMEM((cpw, CHUNK), jnp.int32),
            pltpu.VMEM((cpw, CHUNK), jnp.int32),
            [pltpu.VMEM((CHUNK, fh), jnp.float32) for _ in range(KBUF)],
            pltpu.VMEM_SHARED((n_acc, fh), jnp.float32),
            pltpu.VMEM_SHARED((n_acc, fh), jnp.float32),
            pltpu.SemaphoreType.DMA((KBUF,)),
            pltpu.SemaphoreType.DMA((KBUF,)),
        ],
        compiler_params=pltpu.CompilerParams(use_tc_tiling_on_sc=False),
    )
    def seg_sum(h_hbm, src_hbm, dst_hbm, out_hbm,
                src_v, dst_v, rows, acc, h_sh, sem_g, sem_s):
        cid = lax.axis_index("c")
        sid = lax.axis_index("s")
        base = sid * cpw

        pltpu.sync_copy(src_hbm.at[pl.ds(base, cpw)], src_v)
        pltpu.sync_copy(dst_hbm.at[pl.ds(base, cpw)], dst_v)
        pltpu.sync_copy(h_hbm.at[cid, pl.ds(sid * stripe, stripe)],
                        acc.at[pl.ds(sid * stripe, stripe)])
        pltpu.sync_copy(h_hbm.at[cid, pl.ds(sid * stripe, stripe)],
                        h_sh.at[pl.ds(sid * stripe, stripe)])

        plsc.subcore_barrier()

        for b in range(KBUF):
            pltpu.async_copy(h_sh.at[src_v.at[b]], rows[b], sem_g.at[b])

        def step(i, b, wait_prev, prefetch):
            pltpu.make_async_copy(
                h_sh.at[src_v.at[i]], rows[b], sem_g.at[b]).wait()
            pltpu.async_copy(rows[b], acc.at[dst_v.at[i]], sem_s.at[b],
                             add=True)
            pb = (b - PLAG) % KBUF
            if wait_prev:
                pltpu.make_async_copy(
                    rows[pb], acc.at[dst_v.at[i]], sem_s.at[pb]).wait()
            if prefetch:
                pltpu.async_copy(h_sh.at[src_v.at[i - PLAG + KBUF]],
                                 rows[pb], sem_g.at[pb])

        for b in range(KBUF):
            step(b, b, wait_prev=b >= PLAG, prefetch=b >= PLAG)

        def body(k, carry):
            for b in range(KBUF):
                step(k * KBUF + b, b, wait_prev=True, prefetch=True)
            return carry
        lax.fori_loop(1, cpw // KBUF - 1, body, 0)

        for b in range(KBUF):
            i = cpw - KBUF + b
            step(i, b, wait_prev=True, prefetch=b < PLAG)

        for j in range(cpw - PLAG, cpw):
            b = j % KBUF
            pltpu.make_async_copy(
                rows[b], acc.at[dst_v.at[0]], sem_s.at[b]).wait()

        plsc.subcore_barrier()
        pltpu.sync_copy(acc.at[pl.ds(sid * stripe, stripe)],
                        out_hbm.at[cid, pl.ds(sid * stripe, stripe)])

    return seg_sum, n_acc, stripe


def _linear_relu_body(trash_row, x_ref, w_ref, b_ref, e_ref,
                      h2_ref, src_ref, dst_ref):
    h = jnp.maximum(
        jnp.dot(x_ref[...], w_ref[...], preferred_element_type=jnp.float32)
        + b_ref[...], 0.0)
    n, f = h.shape
    h2_ref[0, :n, :] = h[:, :f // 2]
    h2_ref[1, :n, :] = h[:, f // 2:]
    nr = e_ref.shape[1] // 128
    er = e_ref[...].reshape(2, nr, 128)
    src_ref[:nr] = er[0]
    src_ref[nr:] = jnp.zeros_like(src_ref[nr:])
    dst_ref[:nr] = er[1]
    dst_ref[nr:] = jnp.full_like(dst_ref[nr:], trash_row)


def _mlp_body(n_nodes, p_ref, w1_ref, b1_ref, w2_ref, b2_ref, h2_ref):
    z = jnp.concatenate([p_ref[0, :n_nodes, :], p_ref[1, :n_nodes, :]],
                        axis=1)
    a = jnp.maximum(
        jnp.dot(z, w1_ref[...], preferred_element_type=jnp.float32)
        + b1_ref[...], 0.0)
    h = jnp.maximum(
        jnp.dot(a, w2_ref[...], preferred_element_type=jnp.float32)
        + b2_ref[...], 0.0)
    n, f = h.shape
    h2_ref[0, :n, :] = h[:, :f // 2]
    h2_ref[1, :n, :] = h[:, f // 2:]


def _mlp_pool_body(n_nodes, p_ref, w1_ref, b1_ref, w2_ref, b2_ref,
                   batch_ref, o_ref):
    z = jnp.concatenate([p_ref[0, :n_nodes, :], p_ref[1, :n_nodes, :]],
                        axis=1)
    a = jnp.maximum(
        jnp.dot(z, w1_ref[...], preferred_element_type=jnp.float32)
        + b1_ref[...], 0.0)
    h2 = jnp.maximum(
        jnp.dot(a, w2_ref[...], preferred_element_type=jnp.float32)
        + b2_ref[...], 0.0)
    gids = lax.broadcasted_iota(jnp.int32, (n_nodes, G), 1)
    onehot = (batch_ref[...] == gids).astype(jnp.float32)
    sums = lax.dot_general(onehot, h2, (((0,), (0,)), ((), ())),
                           preferred_element_type=jnp.float32)
    counts = jnp.sum(onehot, axis=0)
    o_ref[...] = sums / jnp.maximum(counts, 1.0)[:, None]


def kernel(x, edge_index, batch, W_in, b_in,
           W1_0, b1_0, W2_0, b2_0, W1_1, b1_1, W2_1, b2_1):
    x = x.astype(jnp.float32)
    n, _ = x.shape
    h_dim = W_in.shape[1]
    e = edge_index.shape[1]

    per_tile = NS * KBUF * CHUNK
    e_pad = -(-e // per_tile) * per_tile
    n_chunk_rows = e_pad // CHUNK
    seg_sum, n_acc, stripe = _make_sc_segment_sum(n, h_dim, n_chunk_rows)
    fh = h_dim // NC

    batch2d = batch.reshape(n, 1)
    b_in2 = b_in.reshape(1, -1)
    b1_02, b2_02 = b1_0.reshape(1, -1), b2_0.reshape(1, -1)
    b1_12, b2_12 = b1_1.reshape(1, -1), b2_1.reshape(1, -1)

    h_split = jax.ShapeDtypeStruct((NC, n_acc, fh), jnp.float32)

    h0s, src2d, dst2d = pl.pallas_call(
        functools.partial(_linear_relu_body, n_acc - 1),
        out_shape=(h_split,
                   jax.ShapeDtypeStruct((n_chunk_rows, CHUNK), jnp.int32),
                   jax.ShapeDtypeStruct((n_chunk_rows, CHUNK), jnp.int32)),
    )(x, W_in, b_in2, edge_index)

    z0 = seg_sum(h0s, src2d, dst2d)

    h1s = pl.pallas_call(
        functools.partial(_mlp_body, n), out_shape=h_split,
    )(z0, W1_0, b1_02, W2_0, b2_02)

    z1 = seg_sum(h1s, src2d, dst2d)

    out = pl.pallas_call(
        functools.partial(_mlp_pool_body, n),
        out_shape=jax.ShapeDtypeStruct((G, h_dim), jnp.float32),
    )(z1, W1_1, b1_12, W2_1, b2_12, batch2d)

    return out


# speedup vs baseline: 1.1225x; 1.0184x over previous
import functools

import jax
import jax.numpy as jnp
from jax import lax
from jax.experimental import pallas as pl
from jax.experimental.pallas import tpu as pltpu
from jax.experimental.pallas import tpu_sc as plsc

G = 64
NC = 2
NS = 16
CHUNK = 128
KBUF = 8
PLAG = 4


def _make_sc_segment_sum(n_nodes, feat, n_chunk_rows):
    fh = feat // NC
    cpw = n_chunk_rows // NS
    assert cpw % KBUF == 0 and cpw >= 2 * KBUF
    stripe = ((n_nodes + NS - 1) // NS + 7) // 8 * 8
    n_acc = NS * stripe
    mesh = plsc.VectorSubcoreMesh(core_axis_name="c", subcore_axis_name="s")

    @functools.partial(
        pl.kernel,
        out_type=jax.ShapeDtypeStruct((NC, n_acc, fh), jnp.float32),
        mesh=mesh,
        scratch_types=[
            pltpu.VMEM((cpw, CHUNK), jnp.int32),
            pltpu.VMEM((cpw, CHUNK), jnp.int32),
            [pltpu.VMEM((CHUNK, fh), jnp.float32) for _ in range(KBUF)],
            pltpu.VMEM_SHARED((n_acc, fh), jnp.float32),
            pltpu.VMEM_SHARED((n_acc, fh), jnp.float32),
            pltpu.SemaphoreType.DMA((KBUF,)),
            pltpu.SemaphoreType.DMA((KBUF,)),
        ],
        compiler_params=pltpu.CompilerParams(use_tc_tiling_on_sc=False),
    )
    def seg_sum(h_hbm, src_hbm, dst_hbm, out_hbm,
                src_v, dst_v, rows, acc, h_sh, sem_g, sem_s):
        cid = lax.axis_index("c")
        sid = lax.axis_index("s")
        base = sid * cpw

        stage = [
            (src_hbm.at[pl.ds(base, cpw)], src_v),
            (dst_hbm.at[pl.ds(base, cpw)], dst_v),
            (h_hbm.at[cid, pl.ds(sid * stripe, stripe)],
             acc.at[pl.ds(sid * stripe, stripe)]),
            (h_hbm.at[cid, pl.ds(sid * stripe, stripe)],
             h_sh.at[pl.ds(sid * stripe, stripe)]),
        ]
        for b, (s_ref, d_ref) in enumerate(stage):
            pltpu.async_copy(s_ref, d_ref, sem_g.at[b])
        for b, (s_ref, d_ref) in enumerate(stage):
            pltpu.make_async_copy(s_ref, d_ref, sem_g.at[b]).wait()

        plsc.subcore_barrier()

        for b in range(KBUF):
            pltpu.async_copy(h_sh.at[src_v.at[b]], rows[b], sem_g.at[b])

        def step(i, b, wait_prev, prefetch):
            pltpu.make_async_copy(
                h_sh.at[src_v.at[i]], rows[b], sem_g.at[b]).wait()
            pltpu.async_copy(rows[b], acc.at[dst_v.at[i]], sem_s.at[b],
                             add=True)
            pb = (b - PLAG) % KBUF
            if wait_prev:
                pltpu.make_async_copy(
                    rows[pb], acc.at[dst_v.at[i]], sem_s.at[pb]).wait()
            if prefetch:
                pltpu.async_copy(h_sh.at[src_v.at[i - PLAG + KBUF]],
                                 rows[pb], sem_g.at[pb])

        for b in range(KBUF):
            step(b, b, wait_prev=b >= PLAG, prefetch=b >= PLAG)

        def body(k, carry):
            for b in range(KBUF):
                step(k * KBUF + b, b, wait_prev=True, prefetch=True)
            return carry
        lax.fori_loop(1, cpw // KBUF - 1, body, 0)

        for b in range(KBUF):
            i = cpw - KBUF + b
            step(i, b, wait_prev=True, prefetch=b < PLAG)

        for j in range(cpw - PLAG, cpw):
            b = j % KBUF
            pltpu.make_async_copy(
                rows[b], acc.at[dst_v.at[0]], sem_s.at[b]).wait()

        plsc.subcore_barrier()
        pltpu.sync_copy(acc.at[pl.ds(sid * stripe, stripe)],
                        out_hbm.at[cid, pl.ds(sid * stripe, stripe)])

    return seg_sum, n_acc, stripe


def _linear_relu_body(trash_row, x_ref, w_ref, b_ref, e_ref,
                      h2_ref, src_ref, dst_ref):
    h = jnp.maximum(
        jnp.dot(x_ref[...], w_ref[...], preferred_element_type=jnp.float32)
        + b_ref[...], 0.0)
    n, f = h.shape
    h2_ref[0, :n, :] = h[:, :f // 2]
    h2_ref[1, :n, :] = h[:, f // 2:]
    nr = e_ref.shape[1] // 128
    er = e_ref[...].reshape(2, nr, 128)
    src_ref[:nr] = er[0]
    src_ref[nr:] = jnp.zeros_like(src_ref[nr:])
    dst_ref[:nr] = er[1]
    dst_ref[nr:] = jnp.full_like(dst_ref[nr:], trash_row)


def _mlp_body(n_nodes, p_ref, w1_ref, b1_ref, w2_ref, b2_ref, h2_ref):
    z = jnp.concatenate([p_ref[0, :n_nodes, :], p_ref[1, :n_nodes, :]],
                        axis=1)
    a = jnp.maximum(
        jnp.dot(z, w1_ref[...], preferred_element_type=jnp.float32)
        + b1_ref[...], 0.0)
    h = jnp.maximum(
        jnp.dot(a, w2_ref[...], preferred_element_type=jnp.float32)
        + b2_ref[...], 0.0)
    n, f = h.shape
    h2_ref[0, :n, :] = h[:, :f // 2]
    h2_ref[1, :n, :] = h[:, f // 2:]


def _mlp_pool_body(n_nodes, p_ref, w1_ref, b1_ref, w2_ref, b2_ref,
                   batch_ref, o_ref):
    z = jnp.concatenate([p_ref[0, :n_nodes, :], p_ref[1, :n_nodes, :]],
                        axis=1)
    a = jnp.maximum(
        jnp.dot(z, w1_ref[...], preferred_element_type=jnp.float32)
        + b1_ref[...], 0.0)
    h2 = jnp.maximum(
        jnp.dot(a, w2_ref[...], preferred_element_type=jnp.float32)
        + b2_ref[...], 0.0)
    gids = lax.broadcasted_iota(jnp.int32, (n_nodes, G), 1)
    onehot = (batch_ref[...] == gids).astype(jnp.float32)
    sums = lax.dot_general(onehot, h2, (((0,), (0,)), ((), ())),
                           preferred_element_type=jnp.float32)
    counts = jnp.sum(onehot, axis=0)
    o_ref[...] = sums / jnp.maximum(counts, 1.0)[:, None]


def kernel(x, edge_index, batch, W_in, b_in,
           W1_0, b1_0, W2_0, b2_0, W1_1, b1_1, W2_1, b2_1):
    x = x.astype(jnp.float32)
    n, _ = x.shape
    h_dim = W_in.shape[1]
    e = edge_index.shape[1]

    per_tile = NS * KBUF * CHUNK
    e_pad = -(-e // per_tile) * per_tile
    n_chunk_rows = e_pad // CHUNK
    seg_sum, n_acc, stripe = _make_sc_segment_sum(n, h_dim, n_chunk_rows)
    fh = h_dim // NC

    batch2d = batch.reshape(n, 1)
    b_in2 = b_in.reshape(1, -1)
    b1_02, b2_02 = b1_0.reshape(1, -1), b2_0.reshape(1, -1)
    b1_12, b2_12 = b1_1.reshape(1, -1), b2_1.reshape(1, -1)

    h_split = jax.ShapeDtypeStruct((NC, n_acc, fh), jnp.float32)

    h0s, src2d, dst2d = pl.pallas_call(
        functools.partial(_linear_relu_body, n_acc - 1),
        out_shape=(h_split,
                   jax.ShapeDtypeStruct((n_chunk_rows, CHUNK), jnp.int32),
                   jax.ShapeDtypeStruct((n_chunk_rows, CHUNK), jnp.int32)),
    )(x, W_in, b_in2, edge_index)

    z0 = seg_sum(h0s, src2d, dst2d)

    h1s = pl.pallas_call(
        functools.partial(_mlp_body, n), out_shape=h_split,
    )(z0, W1_0, b1_02, W2_0, b2_02)

    z1 = seg_sum(h1s, src2d, dst2d)

    out = pl.pallas_call(
        functools.partial(_mlp_pool_body, n),
        out_shape=jax.ShapeDtypeStruct((G, h_dim), jnp.float32),
    )(z1, W1_1, b1_12, W2_1, b2_12, batch2d)

    return out
